# baseline (speedup 1.0000x reference)
import functools

import jax
import jax.numpy as jnp
from jax import lax
from jax.experimental import pallas as pl
from jax.experimental.pallas import tpu as pltpu
from jax.experimental.pallas import tpu_sc as plsc

S = 8192
B = 4
D = 1024
NC = 2
NSUB = 16
NW = NC * NSUB
S_PER_W = S // NW
NS_CHUNK = 8
ROWS = NS_CHUNK * B
SLOT = ROWS + NS_CHUNK
N_CHUNKS = S_PER_W // NS_CHUNK
LANES = 16
NBUF = 3

_mesh = plsc.VectorSubcoreMesh(core_axis_name="c", subcore_axis_name="s")


@functools.partial(
    pl.kernel,
    mesh=_mesh,
    out_type=jax.ShapeDtypeStruct((S, B, D), jnp.float32),
    scratch_types=[
        pltpu.VMEM((B, S_PER_W), jnp.int32),
        pltpu.VMEM((NBUF, SLOT, D), jnp.float32),
        pltpu.SemaphoreType.DMA((NBUF,)),
        pltpu.SemaphoreType.DMA((NBUF,)),
    ],
)
def _embed(x_hbm, tok_hbm, pos_hbm, out_hbm, idx_v, buf_v, gsem, osem):
    wid = lax.axis_index("s") * NC + lax.axis_index("c")
    sbase = wid * S_PER_W
    for bb in range(B):
        pltpu.sync_copy(x_hbm.at[pl.ds(bb * S + sbase, S_PER_W)],
                        idx_v.at[bb])

    def in_issue(g):
        b = lax.rem(g, NBUF)
        for bb in range(B):
            pltpu.async_copy(
                tok_hbm.at[idx_v.at[bb, pl.ds(g * NS_CHUNK, NS_CHUNK)]],
                buf_v.at[b, pl.ds(bb * NS_CHUNK, NS_CHUNK)], gsem.at[b])
        pltpu.async_copy(
            pos_hbm.at[pl.ds(sbase + g * NS_CHUNK, NS_CHUNK)],
            buf_v.at[b, pl.ds(ROWS, NS_CHUNK)], gsem.at[b])

    def in_wait(g):
        b = lax.rem(g, NBUF)
        pltpu.make_async_copy(
            tok_hbm.at[pl.ds(0, SLOT)], buf_v.at[b], gsem.at[b]).wait()

    def out_issue(g):
        b = lax.rem(g, NBUF)
        for bb in range(B):
            pltpu.async_copy(
                buf_v.at[b, pl.ds(bb * NS_CHUNK, NS_CHUNK)],
                out_hbm.at[pl.ds(sbase + g * NS_CHUNK, NS_CHUNK), bb],
                osem.at[b])

    def out_wait(g):
        b = lax.rem(g, NBUF)
        pltpu.make_async_copy(
            buf_v.at[b, pl.ds(0, ROWS)],
            out_hbm.at[pl.ds(sbase + g * NS_CHUNK, NS_CHUNK)],
            osem.at[b]).wait()

    def add_chunk(g):
        b = lax.rem(g, NBUF)
        buf_s = buf_v.at[b]

        def col(c, c3):
            sl = pl.ds(c * LANES, LANES)
            for i in range(NS_CHUNK):
                p = buf_s[ROWS + i, sl]
                for bb in range(B):
                    buf_s[bb * NS_CHUNK + i, sl] += p
            return c3

        lax.fori_loop(0, D // LANES, col, 0, unroll=4)

    for g in range(NBUF - 1):
        in_issue(g)

    UNROLL = 8

    def body(j, carry):
        for u in range(UNROLL):
            g = j * UNROLL + u

            in_wait(g)
            add_chunk(g)
            out_issue(g)

            @pl.when(jnp.logical_and(g + NBUF - 1 < N_CHUNKS, g >= 1))
            def _():
                out_wait(g - 1)

            @pl.when(g + NBUF - 1 < N_CHUNKS)
            def _():
                in_issue(g + NBUF - 1)
        return carry

    lax.fori_loop(0, N_CHUNKS // UNROLL, body, 0)
    for g in range(N_CHUNKS - NBUF, N_CHUNKS):
        out_wait(g)


def kernel(x, token_table, pos_table):
    xt_flat = x.T.reshape(-1)
    out = _embed(xt_flat, token_table, pos_table)
    return out, x.shape[0]


# speedup vs baseline: 1.1880x; 1.1880x over previous
import functools

import jax
import jax.numpy as jnp
from jax import lax
from jax.experimental import pallas as pl
from jax.experimental.pallas import tpu as pltpu
from jax.experimental.pallas import tpu_sc as plsc

S = 8192
B = 4
D = 1024
NC = 2
NSUB = 16
NW = NC * NSUB
S_PER_W = S // NW
NS_CHUNK = 8
ROWS = NS_CHUNK * B
SLOT = ROWS + NS_CHUNK
N_CHUNKS = S_PER_W // NS_CHUNK
LANES = 16
NBUF = 3

_mesh = plsc.VectorSubcoreMesh(core_axis_name="c", subcore_axis_name="s")


@functools.partial(
    pl.kernel,
    mesh=_mesh,
    out_type=jax.ShapeDtypeStruct((S, B, D), jnp.float32),
    scratch_types=[
        pltpu.VMEM((B, S_PER_W), jnp.int32),
        pltpu.VMEM((NBUF, SLOT, D), jnp.float32),
        pltpu.SemaphoreType.DMA((NBUF,)),
        pltpu.SemaphoreType.DMA((NBUF,)),
    ],
)
def _embed(x_hbm, tok_hbm, pos_hbm, out_hbm, idx_v, buf_v, gsem, osem):
    wid = lax.axis_index("s") * NC + lax.axis_index("c")
    sbase = wid * S_PER_W
    for bb in range(B):
        pltpu.sync_copy(x_hbm.at[pl.ds(bb * S + sbase, S_PER_W)],
                        idx_v.at[bb])

    def in_issue(g):
        b = lax.rem(g, NBUF)
        for bb in range(B):
            pltpu.async_copy(
                tok_hbm.at[idx_v.at[bb, pl.ds(g * NS_CHUNK, NS_CHUNK)]],
                buf_v.at[b, pl.ds(bb * NS_CHUNK, NS_CHUNK)], gsem.at[b])
        pltpu.async_copy(
            pos_hbm.at[pl.ds(sbase + g * NS_CHUNK, NS_CHUNK)],
            buf_v.at[b, pl.ds(ROWS, NS_CHUNK)], gsem.at[b])

    def in_wait(g):
        b = lax.rem(g, NBUF)
        pltpu.make_async_copy(
            tok_hbm.at[pl.ds(0, SLOT)], buf_v.at[b], gsem.at[b]).wait()

    def out_issue(g):
        b = lax.rem(g, NBUF)
        for bb in range(B):
            pltpu.async_copy(
                buf_v.at[b, pl.ds(bb * NS_CHUNK, NS_CHUNK)],
                out_hbm.at[pl.ds(sbase + g * NS_CHUNK, NS_CHUNK), bb],
                osem.at[b])

    def out_wait(g):
        b = lax.rem(g, NBUF)
        pltpu.make_async_copy(
            buf_v.at[b, pl.ds(0, ROWS)],
            out_hbm.at[pl.ds(sbase + g * NS_CHUNK, NS_CHUNK)],
            osem.at[b]).wait()

    def add_chunk(g):
        b = lax.rem(g, NBUF)
        buf_s = buf_v.at[b]

        def col(c, c3):
            sl = pl.ds(c * LANES, LANES)
            for i in range(NS_CHUNK):
                p = buf_s[ROWS + i, sl]
                for bb in range(B):
                    buf_s[bb * NS_CHUNK + i, sl] += p
            return c3

        lax.fori_loop(0, D // LANES, col, 0, unroll=2)

    for g in range(NBUF - 1):
        in_issue(g)

    UNROLL = 2

    def body(j, carry):
        for u in range(UNROLL):
            g = j * UNROLL + u

            in_wait(g)
            add_chunk(g)
            out_issue(g)

            @pl.when(jnp.logical_and(g + NBUF - 1 < N_CHUNKS, g >= 1))
            def _():
                out_wait(g - 1)

            @pl.when(g + NBUF - 1 < N_CHUNKS)
            def _():
                in_issue(g + NBUF - 1)
        return carry

    lax.fori_loop(0, N_CHUNKS // UNROLL, body, 0)
    for g in range(N_CHUNKS - NBUF, N_CHUNKS):
        out_wait(g)


def kernel(x, token_table, pos_table):
    xt_flat = x.T.reshape(-1)
    out = _embed(xt_flat, token_table, pos_table)
    return out, x.shape[0]


# speedup vs baseline: 1.1969x; 1.0075x over previous
import functools

import jax
import jax.numpy as jnp
from jax import lax
from jax.experimental import pallas as pl
from jax.experimental.pallas import tpu as pltpu
from jax.experimental.pallas import tpu_sc as plsc

S = 8192
B = 4
D = 1024
NC = 2
NSUB = 16
NW = NC * NSUB
S_PER_W = S // NW
NS_CHUNK = 8
ROWS = NS_CHUNK * B
SLOT = ROWS + NS_CHUNK
N_CHUNKS = S_PER_W // NS_CHUNK
LANES = 16
NBUF = 3

_mesh = plsc.VectorSubcoreMesh(core_axis_name="c", subcore_axis_name="s")


@functools.partial(
    pl.kernel,
    mesh=_mesh,
    out_type=jax.ShapeDtypeStruct((S, B, D), jnp.float32),
    scratch_types=[
        pltpu.VMEM((B, S_PER_W), jnp.int32),
        pltpu.VMEM((NBUF, SLOT, D), jnp.float32),
        pltpu.SemaphoreType.DMA((NBUF,)),
        pltpu.SemaphoreType.DMA((NBUF,)),
    ],
)
def _embed(x_hbm, tok_hbm, pos_hbm, out_hbm, idx_v, buf_v, gsem, osem):
    wid = lax.axis_index("s") * NC + lax.axis_index("c")
    sbase = wid * S_PER_W
    for bb in range(B):
        pltpu.sync_copy(x_hbm.at[pl.ds(bb * S + sbase, S_PER_W)],
                        idx_v.at[bb])

    def in_issue(g):
        b = lax.rem(g, NBUF)
        for bb in range(B):
            pltpu.async_copy(
                tok_hbm.at[idx_v.at[bb, pl.ds(g * NS_CHUNK, NS_CHUNK)]],
                buf_v.at[b, pl.ds(bb * NS_CHUNK, NS_CHUNK)], gsem.at[b])
        pltpu.async_copy(
            pos_hbm.at[pl.ds(sbase + g * NS_CHUNK, NS_CHUNK)],
            buf_v.at[b, pl.ds(ROWS, NS_CHUNK)], gsem.at[b])

    def in_wait(g):
        b = lax.rem(g, NBUF)
        pltpu.make_async_copy(
            tok_hbm.at[pl.ds(0, SLOT)], buf_v.at[b], gsem.at[b]).wait()

    def out_issue(g):
        b = lax.rem(g, NBUF)
        for bb in range(B):
            pltpu.async_copy(
                buf_v.at[b, pl.ds(bb * NS_CHUNK, NS_CHUNK)],
                out_hbm.at[pl.ds(sbase + g * NS_CHUNK, NS_CHUNK), bb],
                osem.at[b])

    def out_wait(g):
        b = lax.rem(g, NBUF)
        pltpu.make_async_copy(
            buf_v.at[b, pl.ds(0, ROWS)],
            out_hbm.at[pl.ds(sbase + g * NS_CHUNK, NS_CHUNK)],
            osem.at[b]).wait()

    def add_chunk(g):
        b = lax.rem(g, NBUF)
        buf_s = buf_v.at[b]

        def col(c, c3):
            sl = pl.ds(c * LANES, LANES)
            for i in range(NS_CHUNK):
                p = buf_s[ROWS + i, sl]
                for bb in range(B):
                    buf_s[bb * NS_CHUNK + i, sl] += p
            return c3

        lax.fori_loop(0, D // LANES, col, 0, unroll=2)

    for g in range(NBUF - 1):
        in_issue(g)

    UNROLL = 1

    def body(j, carry):
        for u in range(UNROLL):
            g = j * UNROLL + u

            in_wait(g)
            add_chunk(g)
            out_issue(g)

            @pl.when(jnp.logical_and(g + NBUF - 1 < N_CHUNKS, g >= 1))
            def _():
                out_wait(g - 1)

            @pl.when(g + NBUF - 1 < N_CHUNKS)
            def _():
                in_issue(g + NBUF - 1)
        return carry

    lax.fori_loop(0, N_CHUNKS // UNROLL, body, 0)
    for g in range(N_CHUNKS - NBUF, N_CHUNKS):
        out_wait(g)


def kernel(x, token_table, pos_table):
    xt_flat = x.T.reshape(-1)
    out = _embed(xt_flat, token_table, pos_table)
    return out, x.shape[0]


# speedup vs baseline: 1.2081x; 1.0093x over previous
import functools

import jax
import jax.numpy as jnp
from jax import lax
from jax.experimental import pallas as pl
from jax.experimental.pallas import tpu as pltpu
from jax.experimental.pallas import tpu_sc as plsc

S = 8192
B = 4
D = 1024
NC = 2
NSUB = 16
NW = NC * NSUB
S_PER_W = S // NW
NS_CHUNK = 8
ROWS = NS_CHUNK * B
SLOT = ROWS + NS_CHUNK
N_CHUNKS = S_PER_W // NS_CHUNK
LANES = 16
NBUF = 3

_mesh = plsc.VectorSubcoreMesh(core_axis_name="c", subcore_axis_name="s")


@functools.partial(
    pl.kernel,
    mesh=_mesh,
    out_type=jax.ShapeDtypeStruct((S, B, D), jnp.float32),
    scratch_types=[
        pltpu.VMEM((B, S_PER_W), jnp.int32),
        pltpu.VMEM((NBUF, SLOT, D), jnp.float32),
        pltpu.SemaphoreType.DMA((NBUF,)),
        pltpu.SemaphoreType.DMA((NBUF,)),
    ],
)
def _embed(x_hbm, tok_hbm, pos_hbm, out_hbm, idx_v, buf_v, gsem, osem):
    wid = lax.axis_index("s") * NC + lax.axis_index("c")
    sbase = wid * S_PER_W
    for bb in range(B):
        pltpu.sync_copy(x_hbm.at[pl.ds(bb * S + sbase, S_PER_W)],
                        idx_v.at[bb])

    def in_issue(g):
        b = lax.rem(g, NBUF)
        for bb in range(B):
            pltpu.async_copy(
                tok_hbm.at[idx_v.at[bb, pl.ds(g * NS_CHUNK, NS_CHUNK)]],
                buf_v.at[b, pl.ds(bb * NS_CHUNK, NS_CHUNK)], gsem.at[b])
        pltpu.async_copy(
            pos_hbm.at[pl.ds(sbase + g * NS_CHUNK, NS_CHUNK)],
            buf_v.at[b, pl.ds(ROWS, NS_CHUNK)], gsem.at[b])

    def in_wait(g):
        b = lax.rem(g, NBUF)
        pltpu.make_async_copy(
            tok_hbm.at[pl.ds(0, SLOT)], buf_v.at[b], gsem.at[b]).wait()

    def out_issue(g):
        b = lax.rem(g, NBUF)
        for bb in range(B):
            pltpu.async_copy(
                buf_v.at[b, pl.ds(bb * NS_CHUNK, NS_CHUNK)],
                out_hbm.at[pl.ds(sbase + g * NS_CHUNK, NS_CHUNK), bb],
                osem.at[b])

    def out_wait(g):
        b = lax.rem(g, NBUF)
        pltpu.make_async_copy(
            buf_v.at[b, pl.ds(0, ROWS)],
            out_hbm.at[pl.ds(sbase + g * NS_CHUNK, NS_CHUNK)],
            osem.at[b]).wait()

    def add_chunk(g):
        b = lax.rem(g, NBUF)
        buf_s = buf_v.at[b]

        def col(c, c3):
            sl = pl.ds(c * LANES, LANES)
            for i in range(NS_CHUNK):
                p = buf_s[ROWS + i, sl]
                for bb in range(B):
                    buf_s[bb * NS_CHUNK + i, sl] += p
            return c3

        lax.fori_loop(0, D // LANES, col, 0)

    for g in range(NBUF - 1):
        in_issue(g)

    UNROLL = 1

    def body(j, carry):
        for u in range(UNROLL):
            g = j * UNROLL + u

            in_wait(g)
            add_chunk(g)
            out_issue(g)

            @pl.when(jnp.logical_and(g + NBUF - 1 < N_CHUNKS, g >= 1))
            def _():
                out_wait(g - 1)

            @pl.when(g + NBUF - 1 < N_CHUNKS)
            def _():
                in_issue(g + NBUF - 1)
        return carry

    lax.fori_loop(0, N_CHUNKS // UNROLL, body, 0)
    for g in range(N_CHUNKS - NBUF, N_CHUNKS):
        out_wait(g)


def kernel(x, token_table, pos_table):
    xt_flat = x.T.reshape(-1)
    out = _embed(xt_flat, token_table, pos_table)
    return out, x.shape[0]
